# baseline (speedup 1.0000x reference)
import jax
import jax.numpy as jnp
from jax import lax
from jax.experimental import pallas as pl
from jax.experimental.pallas import tpu as pltpu
from jax.experimental.pallas import tpu_sc as plsc

NC, NS, L = 2, 16, 16
NW = NC * NS
B, T, C = 16384, 200, 12
BW = B // NW
RB = 8
NCHUNK = T // RB
SRC = (0, 0, 0, 1, 1, 1, 2, 2, 2, 2, 2, 2)


def _body(date_hbm, *rest):
    tab_hbm = rest[:C]
    out_hbm = rest[C]
    tab_v = rest[C + 1:2 * C + 1]
    in_a, in_b, out_a, out_b, s_ia, s_ib, s_oa, s_ob = rest[2 * C + 1:]
    wid = lax.axis_index("s") * NC + lax.axis_index("c")
    for c in range(C):
        pltpu.sync_copy(tab_hbm[c], tab_v[c])
    b0 = wid * BW

    def in_cp(j, buf, sem):
        return pltpu.make_async_copy(
            date_hbm.at[:, pl.ds(j * RB, RB), pl.ds(b0, BW)], buf, sem)

    def out_cp(j, buf, sem):
        return pltpu.make_async_copy(
            buf, out_hbm.at[:, pl.ds(j * RB, RB), pl.ds(b0, BW)], sem)

    iota = jnp.arange(L, dtype=jnp.int32)

    def compute(in_v, out_v):
        def row(r, c2):
            for k in range(BW // L):
                sl = pl.ds(k * L, L)
                d = tuple(in_v[s, r, sl] * L + iota for s in range(3))
                for c in range(C):
                    out_v[c, r, sl] = plsc.load_gather(tab_v[c], [d[SRC[c]]])
            return c2

        lax.fori_loop(0, RB, row, 0)

    in_cp(0, in_a, s_ia).start()

    def iter2(jj, carry):
        j = 2 * jj
        in_cp(j, in_a, s_ia).wait()
        in_cp(j + 1, in_b, s_ib).start()

        @pl.when(jj > 0)
        def _():
            out_cp(j - 2, out_a, s_oa).wait()

        compute(in_a, out_a)
        out_cp(j, out_a, s_oa).start()

        in_cp(j + 1, in_b, s_ib).wait()
        in_cp(j + 2, in_a, s_ia).start()

        @pl.when(jj > 0)
        def _():
            out_cp(j - 1, out_b, s_ob).wait()

        compute(in_b, out_b)
        out_cp(j + 1, out_b, s_ob).start()
        return carry

    lax.fori_loop(0, (NCHUNK - 1) // 2, iter2, 0)

    last = NCHUNK - 1
    in_cp(last, in_a, s_ia).wait()
    out_cp(last - 2, out_a, s_oa).wait()
    compute(in_a, out_a)
    out_cp(last, out_a, s_oa).start()
    out_cp(last - 1, out_b, s_ob).wait()
    out_cp(last, out_a, s_oa).wait()


_TAB_LEN = tuple(n * L for n in (16, 16, 16, 8, 8, 8, 32, 32, 32, 32, 32, 32))

_sc_call = pl.kernel(
    _body,
    out_type=jax.ShapeDtypeStruct((C, T, B), jnp.float32),
    mesh=plsc.VectorSubcoreMesh(core_axis_name="c", subcore_axis_name="s"),
    compiler_params=pltpu.CompilerParams(needs_layout_passes=False),
    scratch_types=(
        [pltpu.VMEM((n,), jnp.float32) for n in _TAB_LEN]
        + [
            pltpu.VMEM((3, RB, BW), jnp.int32),
            pltpu.VMEM((3, RB, BW), jnp.int32),
            pltpu.VMEM((C, RB, BW), jnp.float32),
            pltpu.VMEM((C, RB, BW), jnp.float32),
            pltpu.SemaphoreType.DMA,
            pltpu.SemaphoreType.DMA,
            pltpu.SemaphoreType.DMA,
            pltpu.SemaphoreType.DMA,
        ]
    ),
)


@jax.jit
def kernel(date, month_table, weekday_table, day_table):
    datep = jnp.transpose(date.astype(jnp.int32), (2, 1, 0))
    def rep(col, n):
        col = jnp.pad(col, (0, n - col.shape[0]))
        return jnp.broadcast_to(col[:, None], (n, 16)).reshape(-1)

    cols = []
    for c in range(3):
        cols.append(rep(month_table[:, c], 16))
    for c in range(3):
        cols.append(rep(weekday_table[:, c], 8))
    for c in range(6):
        cols.append(rep(day_table[:, c], 32))
    out = _sc_call(datep, *cols)
    return jnp.transpose(out, (2, 1, 0))


# speedup vs baseline: 1.0190x; 1.0190x over previous
import jax
import jax.numpy as jnp
from jax import lax
from jax.experimental import pallas as pl
from jax.experimental.pallas import tpu as pltpu
from jax.experimental.pallas import tpu_sc as plsc

NC, NS, L = 2, 16, 16
NW = NC * NS
B, T, C = 16384, 200, 12
NQ = 4
QW = B // NQ
NRG = NW // NQ
NCHUNK = T // NRG
SRC = (0, 0, 0, 1, 1, 1, 2, 2, 2, 2, 2, 2)


def _body(date_hbm, *rest):
    tab_hbm = rest[:C]
    out_hbm = rest[C]
    tab_v = rest[C + 1:2 * C + 1]
    in_a, in_b, out_a, out_b, s_ia, s_ib, s_oa, s_ob = rest[2 * C + 1:]
    wid = lax.axis_index("s") * NC + lax.axis_index("c")
    for c in range(C):
        pltpu.sync_copy(tab_hbm[c], tab_v[c])
    rg = wid // NQ
    b0 = (wid % NQ) * QW
    r_base = rg * NCHUNK

    def in_cp(j, buf, sem):
        return pltpu.make_async_copy(
            date_hbm.at[:, pl.ds(r_base + j, 1), pl.ds(b0, QW)], buf, sem)

    def out_cp(j, buf, sem):
        return pltpu.make_async_copy(
            buf, out_hbm.at[:, pl.ds(r_base + j, 1), pl.ds(b0, QW)], sem)

    def compute(in_v, out_v):
        def block(m, c2):
            for k in range(32):
                sl = pl.ds(m * 512 + k * L, L)
                d = (in_v[0, 0, sl], in_v[1, 0, sl], in_v[2, 0, sl])
                for c in range(C):
                    out_v[c, 0, sl] = plsc.load_gather(tab_v[c], [d[SRC[c]]])
            return c2

        lax.fori_loop(0, QW // 512, block, 0)

    in_cp(0, in_a, s_ia).start()

    def iter2(jj, carry):
        j = 2 * jj
        in_cp(j, in_a, s_ia).wait()
        in_cp(j + 1, in_b, s_ib).start()

        @pl.when(jj > 0)
        def _():
            out_cp(j - 2, out_a, s_oa).wait()

        compute(in_a, out_a)
        out_cp(j, out_a, s_oa).start()

        in_cp(j + 1, in_b, s_ib).wait()
        in_cp(j + 2, in_a, s_ia).start()

        @pl.when(jj > 0)
        def _():
            out_cp(j - 1, out_b, s_ob).wait()

        compute(in_b, out_b)
        out_cp(j + 1, out_b, s_ob).start()
        return carry

    lax.fori_loop(0, (NCHUNK - 1) // 2, iter2, 0)

    last = NCHUNK - 1
    in_cp(last, in_a, s_ia).wait()
    out_cp(last - 2, out_a, s_oa).wait()
    compute(in_a, out_a)
    out_cp(last, out_a, s_oa).start()
    out_cp(last - 1, out_b, s_ob).wait()
    out_cp(last, out_a, s_oa).wait()


_TAB_LEN = (16, 16, 16, 8, 8, 8, 32, 32, 32, 32, 32, 32)

_sc_call = pl.kernel(
    _body,
    out_type=jax.ShapeDtypeStruct((C, T, B), jnp.float32),
    mesh=plsc.VectorSubcoreMesh(core_axis_name="c", subcore_axis_name="s"),
    compiler_params=pltpu.CompilerParams(needs_layout_passes=False),
    scratch_types=(
        [pltpu.VMEM((n,), jnp.float32) for n in _TAB_LEN]
        + [
            pltpu.VMEM((3, 1, QW), jnp.int32),
            pltpu.VMEM((3, 1, QW), jnp.int32),
            pltpu.VMEM((C, 1, QW), jnp.float32),
            pltpu.VMEM((C, 1, QW), jnp.float32),
            pltpu.SemaphoreType.DMA,
            pltpu.SemaphoreType.DMA,
            pltpu.SemaphoreType.DMA,
            pltpu.SemaphoreType.DMA,
        ]
    ),
)


@jax.jit
def kernel(date, month_table, weekday_table, day_table):
    datep = jnp.transpose(date.astype(jnp.int32), (2, 1, 0))
    cols = []
    for c in range(3):
        cols.append(jnp.pad(month_table[:, c], (0, 3)))
    for c in range(3):
        cols.append(jnp.pad(weekday_table[:, c], (0, 1)))
    for c in range(6):
        cols.append(day_table[:, c])
    out = _sc_call(datep, *cols)
    return jnp.transpose(out, (2, 1, 0))


# speedup vs baseline: 2.7492x; 2.6979x over previous
import jax
import jax.numpy as jnp
from jax import lax
from jax.experimental import pallas as pl
from jax.experimental.pallas import tpu as pltpu
from jax.experimental.pallas import tpu_sc as plsc

NC, NS, L = 2, 16, 16
NW = NC * NS
B, T, C = 16384, 200, 12
NQ = 4
QW = B // NQ
NRG = NW // NQ
NCHUNK = T // NRG
SRC = (0, 0, 0, 1, 1, 1, 2, 2, 2, 2, 2, 2)


def _body(date_hbm, *rest):
    tab_hbm = rest[:C]
    out_hbm = rest[C]
    tab_v = rest[C + 1:2 * C + 1]
    in_a, in_b, out_a, out_b, s_ia, s_ib, s_oa, s_ob = rest[2 * C + 1:]
    wid = lax.axis_index("s") * NC + lax.axis_index("c")
    for c in range(C):
        pltpu.sync_copy(tab_hbm[c], tab_v[c])
    rg = wid // NQ
    b0 = (wid % NQ) * QW
    r_base = rg * NCHUNK

    def in_cp(j, buf, sem):
        return pltpu.make_async_copy(
            date_hbm.at[:, pl.ds(r_base + j, 1), pl.ds(b0, QW)], buf, sem)

    def out_cp(j, buf, sem):
        return pltpu.make_async_copy(
            buf, out_hbm.at[:, pl.ds(r_base + j, 1), pl.ds(b0, QW)], sem)

    def compute(in_v, out_v):
        @plsc.parallel_loop(0, QW // L, 1, unroll=8)
        def _block(g):
            sl = pl.ds(g * L, L)
            d = (in_v[0, 0, sl], in_v[1, 0, sl], in_v[2, 0, sl])
            for c in range(C):
                out_v[c, 0, sl] = plsc.load_gather(tab_v[c], [d[SRC[c]]])

    in_cp(0, in_a, s_ia).start()

    def iter2(jj, carry):
        j = 2 * jj
        in_cp(j, in_a, s_ia).wait()
        in_cp(j + 1, in_b, s_ib).start()

        @pl.when(jj > 0)
        def _():
            out_cp(j - 2, out_a, s_oa).wait()

        compute(in_a, out_a)
        out_cp(j, out_a, s_oa).start()

        in_cp(j + 1, in_b, s_ib).wait()
        in_cp(j + 2, in_a, s_ia).start()

        @pl.when(jj > 0)
        def _():
            out_cp(j - 1, out_b, s_ob).wait()

        compute(in_b, out_b)
        out_cp(j + 1, out_b, s_ob).start()
        return carry

    lax.fori_loop(0, (NCHUNK - 1) // 2, iter2, 0)

    last = NCHUNK - 1
    in_cp(last, in_a, s_ia).wait()
    out_cp(last - 2, out_a, s_oa).wait()
    compute(in_a, out_a)
    out_cp(last, out_a, s_oa).start()
    out_cp(last - 1, out_b, s_ob).wait()
    out_cp(last, out_a, s_oa).wait()


_TAB_LEN = (16, 16, 16, 8, 8, 8, 32, 32, 32, 32, 32, 32)

_sc_call = pl.kernel(
    _body,
    out_type=jax.ShapeDtypeStruct((C, T, B), jnp.float32),
    mesh=plsc.VectorSubcoreMesh(core_axis_name="c", subcore_axis_name="s"),
    compiler_params=pltpu.CompilerParams(needs_layout_passes=False),
    scratch_types=(
        [pltpu.VMEM((n,), jnp.float32) for n in _TAB_LEN]
        + [
            pltpu.VMEM((3, 1, QW), jnp.int32),
            pltpu.VMEM((3, 1, QW), jnp.int32),
            pltpu.VMEM((C, 1, QW), jnp.float32),
            pltpu.VMEM((C, 1, QW), jnp.float32),
            pltpu.SemaphoreType.DMA,
            pltpu.SemaphoreType.DMA,
            pltpu.SemaphoreType.DMA,
            pltpu.SemaphoreType.DMA,
        ]
    ),
)


@jax.jit
def kernel(date, month_table, weekday_table, day_table):
    datep = jnp.transpose(date.astype(jnp.int32), (2, 1, 0))
    cols = []
    for c in range(3):
        cols.append(jnp.pad(month_table[:, c], (0, 3)))
    for c in range(3):
        cols.append(jnp.pad(weekday_table[:, c], (0, 1)))
    for c in range(6):
        cols.append(day_table[:, c])
    out = _sc_call(datep, *cols)
    return jnp.transpose(out, (2, 1, 0))


# speedup vs baseline: 2.8667x; 1.0427x over previous
import jax
import jax.numpy as jnp
from jax import lax
from jax.experimental import pallas as pl
from jax.experimental.pallas import tpu as pltpu
from jax.experimental.pallas import tpu_sc as plsc

NC, NS, L = 2, 16, 16
NW = NC * NS
B, T, C = 16384, 200, 12
NQ = 4
QW = B // NQ
NRG = NW // NQ
NCHUNK = T // NRG
SRC = (0, 0, 0, 1, 1, 1, 2, 2, 2, 2, 2, 2)


def _body(date_hbm, *rest):
    tab_hbm = rest[:C]
    out_hbm = rest[C]
    tab_v = rest[C + 1:2 * C + 1]
    in_a, in_b, out_a, out_b, s_ia, s_ib, s_oa, s_ob = rest[2 * C + 1:]
    wid = lax.axis_index("s") * NC + lax.axis_index("c")
    rg = wid // NQ
    b0 = (wid % NQ) * QW
    r_base = rg * NCHUNK

    def in_cp(j, buf, sem):
        return pltpu.make_async_copy(
            date_hbm.at[:, pl.ds(r_base + j, 1), pl.ds(b0, QW)], buf, sem)

    def out_cp(j, buf, sem):
        return pltpu.make_async_copy(
            buf, out_hbm.at[:, pl.ds(r_base + j, 1), pl.ds(b0, QW)], sem)

    def compute(in_v, out_v):
        @plsc.parallel_loop(0, QW // L, 1, unroll=8)
        def _block(g):
            sl = pl.ds(g * L, L)
            d = (in_v[0, 0, sl], in_v[1, 0, sl], in_v[2, 0, sl])
            for c in range(C):
                out_v[c, 0, sl] = plsc.load_gather(tab_v[c], [d[SRC[c]]])

    in_cp(0, in_a, s_ia).start()
    for c in range(C):
        pltpu.make_async_copy(tab_hbm[c], tab_v[c], s_ob).start()
    for c in range(C):
        pltpu.make_async_copy(tab_hbm[c], tab_v[c], s_ob).wait()

    def iter2(jj, carry):
        j = 2 * jj
        in_cp(j, in_a, s_ia).wait()
        in_cp(j + 1, in_b, s_ib).start()

        @pl.when(jj > 0)
        def _():
            out_cp(j - 2, out_a, s_oa).wait()

        compute(in_a, out_a)
        out_cp(j, out_a, s_oa).start()

        in_cp(j + 1, in_b, s_ib).wait()
        in_cp(j + 2, in_a, s_ia).start()

        @pl.when(jj > 0)
        def _():
            out_cp(j - 1, out_b, s_ob).wait()

        compute(in_b, out_b)
        out_cp(j + 1, out_b, s_ob).start()
        return carry

    lax.fori_loop(0, (NCHUNK - 1) // 2, iter2, 0)

    last = NCHUNK - 1
    in_cp(last, in_a, s_ia).wait()
    out_cp(last - 2, out_a, s_oa).wait()
    compute(in_a, out_a)
    out_cp(last, out_a, s_oa).start()
    out_cp(last - 1, out_b, s_ob).wait()
    out_cp(last, out_a, s_oa).wait()


_TAB_LEN = (16, 16, 16, 8, 8, 8, 32, 32, 32, 32, 32, 32)

_sc_call = pl.kernel(
    _body,
    out_type=jax.ShapeDtypeStruct((C, T, B), jnp.float32),
    mesh=plsc.VectorSubcoreMesh(core_axis_name="c", subcore_axis_name="s"),
    compiler_params=pltpu.CompilerParams(needs_layout_passes=False),
    scratch_types=(
        [pltpu.VMEM((n,), jnp.float32) for n in _TAB_LEN]
        + [
            pltpu.VMEM((3, 1, QW), jnp.int32),
            pltpu.VMEM((3, 1, QW), jnp.int32),
            pltpu.VMEM((C, 1, QW), jnp.float32),
            pltpu.VMEM((C, 1, QW), jnp.float32),
            pltpu.SemaphoreType.DMA,
            pltpu.SemaphoreType.DMA,
            pltpu.SemaphoreType.DMA,
            pltpu.SemaphoreType.DMA,
        ]
    ),
)


@jax.jit
def kernel(date, month_table, weekday_table, day_table):
    datep = jnp.transpose(date.astype(jnp.int32), (2, 1, 0))
    cols = []
    for c in range(3):
        cols.append(jnp.pad(month_table[:, c], (0, 3)))
    for c in range(3):
        cols.append(jnp.pad(weekday_table[:, c], (0, 1)))
    for c in range(6):
        cols.append(day_table[:, c])
    out = _sc_call(datep, *cols)
    return jnp.transpose(out, (2, 1, 0))
